# baseline (speedup 1.0000x reference)
import functools

import numpy as np
import jax
import jax.numpy as jnp
from jax.experimental import pallas as pl

NUM_NODES = 64
OBS_DIM = 1024
GNN_OBS = OBS_DIM // NUM_NODES
GNN_ACT = 2
HIDDEN = 128
LOG_STD_MIN, LOG_STD_MAX = -5.0, 2.0

BATCH_BLOCK = 128
CHUNK = 512


def _coeffs(rows):
    n = jax.lax.rem(jax.lax.broadcasted_iota(jnp.int32, (rows, 1), 0),
                    NUM_NODES)
    third = jnp.float32(1.0 / 3.0)
    s6 = jnp.float32(1.0 / np.sqrt(6.0))
    last = NUM_NODES - 1
    av = jnp.where((n == 0) | (n == last), jnp.float32(0.5), third)
    lv = jnp.where(n == 0, jnp.float32(0.0),
                   jnp.where((n == 1) | (n == last), s6, third))
    uv = jnp.where(n == last, jnp.float32(0.0),
                   jnp.where((n == 0) | (n == last - 1), s6, third))
    return av, lv, uv


def _fused_kernel(x_ref, w0_ref, b0_ref, w1_ref, b1_ref, w2_ref, b2_ref,
                  mu_ref, std_ref):
    rows = x_ref.shape[0]
    av, lv, uv = _coeffs(CHUNK)
    w0 = w0_ref[...]
    b0 = b0_ref[...]
    w1 = w1_ref[...]
    b1 = b1_ref[...]
    w2 = w2_ref[...]
    b2 = b2_ref[...]

    def agg(h):
        prev = jnp.roll(h, 1, axis=0)
        nxt = jnp.roll(h, -1, axis=0)
        return av * h + lv * prev + uv * nxt

    def body(c, carry):
        base = c * CHUNK
        x = x_ref[pl.ds(base, CHUNK), :]
        h = jnp.dot(x, w0, preferred_element_type=jnp.float32)
        h = jax.nn.relu(agg(h) + b0)
        h = jnp.dot(h, w1, preferred_element_type=jnp.float32)
        h = jax.nn.relu(agg(h) + b1)
        g = agg(h)
        h2 = jnp.dot(g, w2, preferred_element_type=jnp.float32) + b2
        mu_ref[pl.ds(base, CHUNK), :] = h2[:, :GNN_ACT]
        ls = jnp.tanh(h2[:, GNN_ACT:])
        ls = LOG_STD_MIN + 0.5 * (LOG_STD_MAX - LOG_STD_MIN) * (ls + 1.0)
        std_ref[pl.ds(base, CHUNK), :] = jnp.exp(ls)
        return carry

    jax.lax.fori_loop(0, rows // CHUNK, body, 0, unroll=4)


@functools.partial(jax.jit, static_argnames=())
def kernel(obs, W0, b0, W1, b1, W2, b2):
    bs = obs.shape[0]
    rows = bs * NUM_NODES
    block_rows = BATCH_BLOCK * NUM_NODES
    out_w = NUM_NODES * GNN_ACT
    grid = (bs // BATCH_BLOCK,)

    x = obs.reshape(rows, GNN_OBS)

    const = lambda shape: pl.BlockSpec(shape, lambda i: (0, 0))
    mu, std = pl.pallas_call(
        _fused_kernel,
        grid=grid,
        in_specs=[
            pl.BlockSpec((block_rows, GNN_OBS), lambda i: (i, 0)),
            const((GNN_OBS, HIDDEN)),
            const((1, HIDDEN)),
            const((HIDDEN, HIDDEN)),
            const((1, HIDDEN)),
            const((HIDDEN, 2 * GNN_ACT)),
            const((1, 2 * GNN_ACT)),
        ],
        out_specs=[
            pl.BlockSpec((block_rows, GNN_ACT), lambda i: (i, 0)),
            pl.BlockSpec((block_rows, GNN_ACT), lambda i: (i, 0)),
        ],
        out_shape=[
            jax.ShapeDtypeStruct((rows, GNN_ACT), jnp.float32),
            jax.ShapeDtypeStruct((rows, GNN_ACT), jnp.float32),
        ],
    )(x, W0, b0.reshape(1, HIDDEN), W1, b1.reshape(1, HIDDEN),
      W2, b2.reshape(1, 2 * GNN_ACT))

    return (mu.reshape(bs, out_w), std.reshape(bs, out_w))


# speedup vs baseline: 1.8240x; 1.8240x over previous
import functools

import numpy as np
import jax
import jax.numpy as jnp
from jax.experimental import pallas as pl

NUM_NODES = 64
OBS_DIM = 1024
GNN_OBS = OBS_DIM // NUM_NODES
GNN_ACT = 2
HIDDEN = 128
LOG_STD_MIN, LOG_STD_MAX = -5.0, 2.0

BATCH_BLOCK = 256
CHUNK_B = 16
KGRP = 8
GSIZE = CHUNK_B * KGRP
CROWS = CHUNK_B * NUM_NODES


def _coeffs_perm():
    r = jax.lax.broadcasted_iota(jnp.int32, (CROWS, 1), 0)
    n = 8 * jax.lax.rem(r, KGRP) + r // GSIZE
    third = jnp.float32(1.0 / 3.0)
    s6 = jnp.float32(1.0 / np.sqrt(6.0))
    last = NUM_NODES - 1
    av = jnp.where((n == 0) | (n == last), jnp.float32(0.5), third)
    lv = jnp.where(n == 0, jnp.float32(0.0),
                   jnp.where((n == 1) | (n == last), s6, third))
    uv = jnp.where(n == last, jnp.float32(0.0),
                   jnp.where((n == 0) | (n == last - 1), s6, third))
    return av, lv, uv


def _fused_kernel(x_ref, w0a_ref, b0_ref, w1_ref, b1_ref, w2_ref, b2_ref,
                  mu_ref, std_ref):
    nb = x_ref.shape[0]
    av, lv, uv = _coeffs_perm()
    w0a = w0a_ref[...]
    b0 = b0_ref[...]
    w1 = w1_ref[...]
    b1 = b1_ref[...]
    w2 = w2_ref[...]
    b2 = b2_ref[...]
    def agg(h):
        prev = jnp.concatenate(
            [jnp.roll(h[(KGRP - 1) * GSIZE:, :], 1, axis=0),
             h[:(KGRP - 1) * GSIZE, :]], axis=0)
        nxt = jnp.concatenate(
            [h[GSIZE:, :],
             jnp.roll(h[:GSIZE, :], -1, axis=0)], axis=0)
        return av * h + lv * prev + uv * nxt

    def body(c, carry):
        xc = x_ref[pl.ds(c * CHUNK_B, CHUNK_B), :, :].reshape(GSIZE, HIDDEN)
        y = jnp.dot(xc, w0a, preferred_element_type=jnp.float32)
        h = jnp.concatenate(
            [y[:, k * HIDDEN:(k + 1) * HIDDEN] for k in range(KGRP)], axis=0)
        h = jax.nn.relu(agg(h) + b0)
        h = jnp.dot(h, w1, preferred_element_type=jnp.float32)
        h = jax.nn.relu(agg(h) + b1)
        g = agg(h)
        h2 = jnp.dot(g, w2, preferred_element_type=jnp.float32) + b2
        base = c * GSIZE
        for k in range(KGRP):
            part = h2[k * GSIZE:(k + 1) * GSIZE, :]
            mu_ref[pl.ds(base, GSIZE), pl.ds(GNN_ACT * k, GNN_ACT)] = (
                part[:, :GNN_ACT])
            std_ref[pl.ds(base, GSIZE), pl.ds(GNN_ACT * k, GNN_ACT)] = (
                part[:, GNN_ACT:])
        return carry

    jax.lax.fori_loop(0, nb // CHUNK_B, body, 0, unroll=2)

    ls = jnp.tanh(std_ref[...])
    ls = LOG_STD_MIN + 0.5 * (LOG_STD_MAX - LOG_STD_MIN) * (ls + 1.0)
    std_ref[...] = jnp.exp(ls)


@functools.partial(jax.jit, static_argnames=())
def kernel(obs, W0, b0, W1, b1, W2, b2):
    bs = obs.shape[0]
    out_w = NUM_NODES * GNN_ACT
    grid = (bs // BATCH_BLOCK,)

    x = obs.reshape(bs, KGRP, HIDDEN)
    W0all = jnp.kron(jnp.eye(KGRP, dtype=jnp.float32), W0)

    ow = KGRP * GNN_ACT
    mu, std = pl.pallas_call(
        _fused_kernel,
        grid=grid,
        in_specs=[
            pl.BlockSpec((BATCH_BLOCK, KGRP, HIDDEN), lambda i: (i, 0, 0)),
            pl.BlockSpec((HIDDEN, KGRP * HIDDEN), lambda i: (0, 0)),
            pl.BlockSpec((1, HIDDEN), lambda i: (0, 0)),
            pl.BlockSpec((HIDDEN, HIDDEN), lambda i: (0, 0)),
            pl.BlockSpec((1, HIDDEN), lambda i: (0, 0)),
            pl.BlockSpec((HIDDEN, 2 * GNN_ACT), lambda i: (0, 0)),
            pl.BlockSpec((1, 2 * GNN_ACT), lambda i: (0, 0)),
        ],
        out_specs=[
            pl.BlockSpec((BATCH_BLOCK * KGRP, ow), lambda i: (i, 0)),
            pl.BlockSpec((BATCH_BLOCK * KGRP, ow), lambda i: (i, 0)),
        ],
        out_shape=[
            jax.ShapeDtypeStruct((bs * KGRP, ow), jnp.float32),
            jax.ShapeDtypeStruct((bs * KGRP, ow), jnp.float32),
        ],
    )(x, W0all, b0.reshape(1, HIDDEN), W1, b1.reshape(1, HIDDEN),
      W2, b2.reshape(1, 2 * GNN_ACT))

    return (mu.reshape(bs, out_w), std.reshape(bs, out_w))


# speedup vs baseline: 1.8968x; 1.0399x over previous
import functools

import numpy as np
import jax
import jax.numpy as jnp
from jax.experimental import pallas as pl

NUM_NODES = 64
OBS_DIM = 1024
GNN_OBS = OBS_DIM // NUM_NODES
GNN_ACT = 2
HIDDEN = 128
LOG_STD_MIN, LOG_STD_MAX = -5.0, 2.0

BATCH_BLOCK = 256
CHUNK_B = 16
KGRP = 8
GSIZE = CHUNK_B * KGRP
CROWS = CHUNK_B * NUM_NODES


THIRD = 1.0 / 3.0
S6 = float(1.0 / np.sqrt(6.0))


def _edge_coeffs():
    j = jax.lax.rem(jax.lax.broadcasted_iota(jnp.int32, (GSIZE, 1), 0),
                    KGRP)
    j0 = j == 0
    j7 = j == KGRP - 1
    third = jnp.float32(THIRD)
    sel = lambda m, v: jnp.where(m, jnp.float32(v), third)
    av0 = sel(j0, 0.5)
    lv0 = sel(j0, 0.0)
    uv0 = sel(j0, S6)
    lv1 = sel(j0, S6)
    uv6 = sel(j7, S6)
    av7 = sel(j7, 0.5)
    lv7 = sel(j7, S6)
    uv7 = sel(j7, 0.0)
    return av0, lv0, uv0, lv1, uv6, av7, lv7, uv7


def _fused_kernel(x_ref, w0a_ref, b0_ref, w1_ref, b1_ref, w2_ref, b2_ref,
                  mu_ref, std_ref):
    nb = x_ref.shape[0]
    av0, lv0, uv0, lv1, uv6, av7, lv7, uv7 = _edge_coeffs()
    w0a = w0a_ref[...]
    b0 = b0_ref[...]
    w1 = w1_ref[...]
    b1 = b1_ref[...]
    w2 = w2_ref[...]
    b2 = b2_ref[...]
    third = jnp.float32(THIRD)

    def agg(h):
        g = [h[k * GSIZE:(k + 1) * GSIZE, :] for k in range(KGRP)]
        prev = [jnp.roll(g[KGRP - 1], 1, axis=0)] + g[:KGRP - 1]
        nxt = g[1:] + [jnp.roll(g[0], -1, axis=0)]
        out = [
            av0 * g[0] + lv0 * prev[0] + uv0 * nxt[0],
            third * (g[1] + nxt[1]) + lv1 * prev[1],
            third * ((g[2] + prev[2]) + nxt[2]),
            third * ((g[3] + prev[3]) + nxt[3]),
            third * ((g[4] + prev[4]) + nxt[4]),
            third * ((g[5] + prev[5]) + nxt[5]),
            third * (g[6] + prev[6]) + uv6 * nxt[6],
            av7 * g[7] + lv7 * prev[7] + uv7 * nxt[7],
        ]
        return jnp.concatenate(out, axis=0)

    def body(c, carry):
        xc = x_ref[pl.ds(c * CHUNK_B, CHUNK_B), :, :].reshape(GSIZE, HIDDEN)
        y = jnp.dot(xc, w0a, preferred_element_type=jnp.float32)
        h = jnp.concatenate(
            [y[:, k * HIDDEN:(k + 1) * HIDDEN] for k in range(KGRP)], axis=0)
        h = jax.nn.relu(agg(h) + b0)
        h = jnp.dot(h, w1, preferred_element_type=jnp.float32)
        h = jax.nn.relu(agg(h) + b1)
        g = agg(h)
        h2 = jnp.dot(g, w2, preferred_element_type=jnp.float32) + b2
        base = c * GSIZE
        for k in range(KGRP):
            part = h2[k * GSIZE:(k + 1) * GSIZE, :]
            mu_ref[pl.ds(base, GSIZE), pl.ds(GNN_ACT * k, GNN_ACT)] = (
                part[:, :GNN_ACT])
            std_ref[pl.ds(base, GSIZE), pl.ds(GNN_ACT * k, GNN_ACT)] = (
                part[:, GNN_ACT:])
        return carry

    jax.lax.fori_loop(0, nb // CHUNK_B, body, 0, unroll=2)

    ls = jnp.tanh(std_ref[...])
    ls = LOG_STD_MIN + 0.5 * (LOG_STD_MAX - LOG_STD_MIN) * (ls + 1.0)
    std_ref[...] = jnp.exp(ls)


@functools.partial(jax.jit, static_argnames=())
def kernel(obs, W0, b0, W1, b1, W2, b2):
    bs = obs.shape[0]
    out_w = NUM_NODES * GNN_ACT
    grid = (bs // BATCH_BLOCK,)

    x = obs.reshape(bs, KGRP, HIDDEN)
    W0all = jnp.kron(jnp.eye(KGRP, dtype=jnp.float32), W0)

    ow = KGRP * GNN_ACT
    mu, std = pl.pallas_call(
        _fused_kernel,
        grid=grid,
        in_specs=[
            pl.BlockSpec((BATCH_BLOCK, KGRP, HIDDEN), lambda i: (i, 0, 0)),
            pl.BlockSpec((HIDDEN, KGRP * HIDDEN), lambda i: (0, 0)),
            pl.BlockSpec((1, HIDDEN), lambda i: (0, 0)),
            pl.BlockSpec((HIDDEN, HIDDEN), lambda i: (0, 0)),
            pl.BlockSpec((1, HIDDEN), lambda i: (0, 0)),
            pl.BlockSpec((HIDDEN, 2 * GNN_ACT), lambda i: (0, 0)),
            pl.BlockSpec((1, 2 * GNN_ACT), lambda i: (0, 0)),
        ],
        out_specs=[
            pl.BlockSpec((BATCH_BLOCK * KGRP, ow), lambda i: (i, 0)),
            pl.BlockSpec((BATCH_BLOCK * KGRP, ow), lambda i: (i, 0)),
        ],
        out_shape=[
            jax.ShapeDtypeStruct((bs * KGRP, ow), jnp.float32),
            jax.ShapeDtypeStruct((bs * KGRP, ow), jnp.float32),
        ],
    )(x, W0all, b0.reshape(1, HIDDEN), W1, b1.reshape(1, HIDDEN),
      W2, b2.reshape(1, 2 * GNN_ACT))

    return (mu.reshape(bs, out_w), std.reshape(bs, out_w))


# speedup vs baseline: 2.0410x; 1.0760x over previous
import functools

import numpy as np
import jax
import jax.numpy as jnp
from jax.experimental import pallas as pl

NUM_NODES = 64
OBS_DIM = 1024
GNN_OBS = OBS_DIM // NUM_NODES
GNN_ACT = 2
HIDDEN = 128
LOG_STD_MIN, LOG_STD_MAX = -5.0, 2.0

BATCH_BLOCK = 256
CHUNK_B = 16
KGRP = 8
GSIZE = CHUNK_B * KGRP
CROWS = CHUNK_B * NUM_NODES


THIRD = 1.0 / 3.0
S6 = float(1.0 / np.sqrt(6.0))


def _edge_coeffs():
    j = jax.lax.rem(jax.lax.broadcasted_iota(jnp.int32, (GSIZE, 1), 0),
                    KGRP)
    j0 = j == 0
    j7 = j == KGRP - 1
    third = jnp.float32(THIRD)
    sel = lambda m, v: jnp.where(m, jnp.float32(v), third)
    av0 = sel(j0, 0.5)
    lv0 = sel(j0, 0.0)
    uv0 = sel(j0, S6)
    lv1 = sel(j0, S6)
    uv6 = sel(j7, S6)
    av7 = sel(j7, 0.5)
    lv7 = sel(j7, S6)
    uv7 = sel(j7, 0.0)
    return av0, lv0, uv0, lv1, uv6, av7, lv7, uv7


def _fused_kernel(x_ref, w0a_ref, b0_ref, w1_ref, b1_ref, w2_ref, b2_ref,
                  mu_ref, std_ref):
    nb = x_ref.shape[0]
    av0, lv0, uv0, lv1, uv6, av7, lv7, uv7 = _edge_coeffs()
    w0a = w0a_ref[...]
    b0 = b0_ref[...]
    w1 = w1_ref[...]
    b1 = b1_ref[...]
    w2 = w2_ref[...]
    b2 = b2_ref[...]
    third = jnp.float32(THIRD)

    def agg(g):
        prev = [jnp.roll(g[KGRP - 1], 1, axis=0)] + g[:KGRP - 1]
        nxt = g[1:] + [jnp.roll(g[0], -1, axis=0)]
        return [
            av0 * g[0] + lv0 * prev[0] + uv0 * nxt[0],
            third * (g[1] + nxt[1]) + lv1 * prev[1],
            third * ((g[2] + prev[2]) + nxt[2]),
            third * ((g[3] + prev[3]) + nxt[3]),
            third * ((g[4] + prev[4]) + nxt[4]),
            third * ((g[5] + prev[5]) + nxt[5]),
            third * (g[6] + prev[6]) + uv6 * nxt[6],
            av7 * g[7] + lv7 * prev[7] + uv7 * nxt[7],
        ]

    def body(c, carry):
        xc = x_ref[pl.ds(c * CHUNK_B, CHUNK_B), :, :].reshape(GSIZE, HIDDEN)
        y = jnp.dot(xc, w0a, preferred_element_type=jnp.float32)
        h = [y[:, k * HIDDEN:(k + 1) * HIDDEN] for k in range(KGRP)]
        h = [jax.nn.relu(t + b0) for t in agg(h)]
        h = [jnp.dot(t, w1, preferred_element_type=jnp.float32) for t in h]
        h = [jax.nn.relu(t + b1) for t in agg(h)]
        g = agg(h)
        base = c * GSIZE
        for k in range(KGRP):
            h2 = jnp.dot(g[k], w2, preferred_element_type=jnp.float32) + b2
            mu_ref[pl.ds(base, GSIZE), pl.ds(GNN_ACT * k, GNN_ACT)] = (
                h2[:, :GNN_ACT])
            std_ref[pl.ds(base, GSIZE), pl.ds(GNN_ACT * k, GNN_ACT)] = (
                h2[:, GNN_ACT:])
        return carry

    jax.lax.fori_loop(0, nb // CHUNK_B, body, 0, unroll=2)

    ls = jnp.tanh(std_ref[...])
    ls = LOG_STD_MIN + 0.5 * (LOG_STD_MAX - LOG_STD_MIN) * (ls + 1.0)
    std_ref[...] = jnp.exp(ls)


@functools.partial(jax.jit, static_argnames=())
def kernel(obs, W0, b0, W1, b1, W2, b2):
    bs = obs.shape[0]
    out_w = NUM_NODES * GNN_ACT
    grid = (bs // BATCH_BLOCK,)

    x = obs.reshape(bs, KGRP, HIDDEN)
    W0all = jnp.kron(jnp.eye(KGRP, dtype=jnp.float32), W0)

    ow = KGRP * GNN_ACT
    mu, std = pl.pallas_call(
        _fused_kernel,
        grid=grid,
        in_specs=[
            pl.BlockSpec((BATCH_BLOCK, KGRP, HIDDEN), lambda i: (i, 0, 0)),
            pl.BlockSpec((HIDDEN, KGRP * HIDDEN), lambda i: (0, 0)),
            pl.BlockSpec((1, HIDDEN), lambda i: (0, 0)),
            pl.BlockSpec((HIDDEN, HIDDEN), lambda i: (0, 0)),
            pl.BlockSpec((1, HIDDEN), lambda i: (0, 0)),
            pl.BlockSpec((HIDDEN, 2 * GNN_ACT), lambda i: (0, 0)),
            pl.BlockSpec((1, 2 * GNN_ACT), lambda i: (0, 0)),
        ],
        out_specs=[
            pl.BlockSpec((BATCH_BLOCK * KGRP, ow), lambda i: (i, 0)),
            pl.BlockSpec((BATCH_BLOCK * KGRP, ow), lambda i: (i, 0)),
        ],
        out_shape=[
            jax.ShapeDtypeStruct((bs * KGRP, ow), jnp.float32),
            jax.ShapeDtypeStruct((bs * KGRP, ow), jnp.float32),
        ],
    )(x, W0all, b0.reshape(1, HIDDEN), W1, b1.reshape(1, HIDDEN),
      W2, b2.reshape(1, 2 * GNN_ACT))

    return (mu.reshape(bs, out_w), std.reshape(bs, out_w))


# speedup vs baseline: 2.1941x; 1.0750x over previous
import functools

import numpy as np
import jax
import jax.numpy as jnp
from jax.experimental import pallas as pl

NUM_NODES = 64
OBS_DIM = 1024
GNN_OBS = OBS_DIM // NUM_NODES
GNN_ACT = 2
HIDDEN = 128
LOG_STD_MIN, LOG_STD_MAX = -5.0, 2.0

BATCH_BLOCK = 256
CHUNK_B = 16
KGRP = 8
GSIZE = CHUNK_B * KGRP
CROWS = CHUNK_B * NUM_NODES


THIRD = 1.0 / 3.0
S6 = float(1.0 / np.sqrt(6.0))


def _edge_coeffs():
    j = jax.lax.rem(jax.lax.broadcasted_iota(jnp.int32, (GSIZE, 1), 0),
                    KGRP)
    j0 = j == 0
    j7 = j == KGRP - 1
    third = jnp.float32(THIRD)
    sel = lambda m, v: jnp.where(m, jnp.float32(v), third)
    av0 = sel(j0, 0.5)
    lv0 = sel(j0, 0.0)
    uv0 = sel(j0, S6)
    lv1 = sel(j0, S6)
    uv6 = sel(j7, S6)
    av7 = sel(j7, 0.5)
    lv7 = sel(j7, S6)
    uv7 = sel(j7, 0.0)
    return av0, lv0, uv0, lv1, uv6, av7, lv7, uv7


def _fused_kernel(x_ref, w0a_ref, b0_ref, w1_ref, b1_ref, w2s_ref, b2s_ref,
                  mu_ref, std_ref):
    nb = x_ref.shape[0]
    av0, lv0, uv0, lv1, uv6, av7, lv7, uv7 = _edge_coeffs()
    w0a = w0a_ref[...]
    b0 = b0_ref[...]
    w1 = w1_ref[...]
    b1 = b1_ref[...]
    w2s = w2s_ref[...]
    b2s = b2s_ref[...]
    third = jnp.float32(THIRD)

    def agg(g):
        prev = [jnp.roll(g[KGRP - 1], 1, axis=0)] + g[:KGRP - 1]
        nxt = g[1:] + [jnp.roll(g[0], -1, axis=0)]
        return [
            av0 * g[0] + lv0 * prev[0] + uv0 * nxt[0],
            third * (g[1] + nxt[1]) + lv1 * prev[1],
            third * ((g[2] + prev[2]) + nxt[2]),
            third * ((g[3] + prev[3]) + nxt[3]),
            third * ((g[4] + prev[4]) + nxt[4]),
            third * ((g[5] + prev[5]) + nxt[5]),
            third * (g[6] + prev[6]) + uv6 * nxt[6],
            av7 * g[7] + lv7 * prev[7] + uv7 * nxt[7],
        ]

    def body(c, carry):
        xc = x_ref[pl.ds(c * CHUNK_B, CHUNK_B), :, :].reshape(GSIZE, HIDDEN)
        y = jnp.dot(xc, w0a, preferred_element_type=jnp.float32)
        h = [y[:, k * HIDDEN:(k + 1) * HIDDEN] for k in range(KGRP)]
        h = [jax.nn.relu(t + b0) for t in agg(h)]
        h = [jnp.dot(t, w1, preferred_element_type=jnp.float32) for t in h]
        h = [jax.nn.relu(t + b1) for t in agg(h)]
        g = agg(h)
        gw = jnp.concatenate(g, axis=1)
        p = jnp.dot(gw, w2s, preferred_element_type=jnp.float32) + b2s
        base = c * GSIZE
        ow = KGRP * GNN_ACT
        mu_ref[pl.ds(base, GSIZE), :] = p[:, :ow]
        std_ref[pl.ds(base, GSIZE), :] = p[:, ow:]
        return carry

    jax.lax.fori_loop(0, nb // CHUNK_B, body, 0, unroll=2)

    ls = jnp.tanh(std_ref[...])
    ls = LOG_STD_MIN + 0.5 * (LOG_STD_MAX - LOG_STD_MIN) * (ls + 1.0)
    std_ref[...] = jnp.exp(ls)


@functools.partial(jax.jit, static_argnames=())
def kernel(obs, W0, b0, W1, b1, W2, b2):
    bs = obs.shape[0]
    out_w = NUM_NODES * GNN_ACT
    grid = (bs // BATCH_BLOCK,)

    x = obs.reshape(bs, KGRP, HIDDEN)
    eye = jnp.eye(KGRP, dtype=jnp.float32)
    W0all = jnp.kron(eye, W0)
    W2s = jnp.concatenate(
        [jnp.kron(eye, W2[:, :GNN_ACT]), jnp.kron(eye, W2[:, GNN_ACT:])],
        axis=1)
    b2s = jnp.concatenate(
        [jnp.tile(b2[:GNN_ACT], KGRP), jnp.tile(b2[GNN_ACT:], KGRP)])

    ow = KGRP * GNN_ACT
    mu, std = pl.pallas_call(
        _fused_kernel,
        grid=grid,
        in_specs=[
            pl.BlockSpec((BATCH_BLOCK, KGRP, HIDDEN), lambda i: (i, 0, 0)),
            pl.BlockSpec((HIDDEN, KGRP * HIDDEN), lambda i: (0, 0)),
            pl.BlockSpec((1, HIDDEN), lambda i: (0, 0)),
            pl.BlockSpec((HIDDEN, HIDDEN), lambda i: (0, 0)),
            pl.BlockSpec((1, HIDDEN), lambda i: (0, 0)),
            pl.BlockSpec((KGRP * HIDDEN, 2 * KGRP * GNN_ACT), lambda i: (0, 0)),
            pl.BlockSpec((1, 2 * KGRP * GNN_ACT), lambda i: (0, 0)),
        ],
        out_specs=[
            pl.BlockSpec((BATCH_BLOCK * KGRP, ow), lambda i: (i, 0)),
            pl.BlockSpec((BATCH_BLOCK * KGRP, ow), lambda i: (i, 0)),
        ],
        out_shape=[
            jax.ShapeDtypeStruct((bs * KGRP, ow), jnp.float32),
            jax.ShapeDtypeStruct((bs * KGRP, ow), jnp.float32),
        ],
    )(x, W0all, b0.reshape(1, HIDDEN), W1, b1.reshape(1, HIDDEN),
      W2s, b2s.reshape(1, 2 * KGRP * GNN_ACT))

    return (mu.reshape(bs, out_w), std.reshape(bs, out_w))


# speedup vs baseline: 2.2249x; 1.0140x over previous
import functools

import numpy as np
import jax
import jax.numpy as jnp
from jax.experimental import pallas as pl

NUM_NODES = 64
OBS_DIM = 1024
GNN_OBS = OBS_DIM // NUM_NODES
GNN_ACT = 2
HIDDEN = 128
LOG_STD_MIN, LOG_STD_MAX = -5.0, 2.0

BATCH_BLOCK = 256
CHUNK_B = 16
KGRP = 8
GSIZE = CHUNK_B * KGRP
CROWS = CHUNK_B * NUM_NODES


THIRD = 1.0 / 3.0
S6 = float(1.0 / np.sqrt(6.0))


def _edge_coeffs():
    j = jax.lax.rem(jax.lax.broadcasted_iota(jnp.int32, (GSIZE, 1), 0),
                    KGRP)
    j0 = j == 0
    j7 = j == KGRP - 1
    third = jnp.float32(THIRD)
    sel = lambda m, v: jnp.where(m, jnp.float32(v), third)
    av0 = sel(j0, 0.5)
    lv0 = sel(j0, 0.0)
    uv0 = sel(j0, S6)
    lv1 = sel(j0, S6)
    uv6 = sel(j7, S6)
    av7 = sel(j7, 0.5)
    lv7 = sel(j7, S6)
    uv7 = sel(j7, 0.0)
    return av0, lv0, uv0, lv1, uv6, av7, lv7, uv7


def _fused_kernel(x_ref, w0a_ref, b0_ref, w1_ref, b1_ref, w2s_ref, b2s_ref,
                  mu_ref, std_ref):
    nb = x_ref.shape[0]
    av0, lv0, uv0, lv1, uv6, av7, lv7, uv7 = _edge_coeffs()
    w0a = w0a_ref[...]
    b0 = b0_ref[...]
    w1 = w1_ref[...]
    b1 = b1_ref[...]
    w2s = w2s_ref[...]
    b2s = b2s_ref[...]
    third = jnp.float32(THIRD)

    def agg(g):
        prev = [jnp.roll(g[KGRP - 1], 1, axis=0)] + g[:KGRP - 1]
        nxt = g[1:] + [jnp.roll(g[0], -1, axis=0)]
        return [
            av0 * g[0] + lv0 * prev[0] + uv0 * nxt[0],
            third * (g[1] + nxt[1]) + lv1 * prev[1],
            third * ((g[2] + prev[2]) + nxt[2]),
            third * ((g[3] + prev[3]) + nxt[3]),
            third * ((g[4] + prev[4]) + nxt[4]),
            third * ((g[5] + prev[5]) + nxt[5]),
            third * (g[6] + prev[6]) + uv6 * nxt[6],
            av7 * g[7] + lv7 * prev[7] + uv7 * nxt[7],
        ]

    def body(c, carry):
        xc = x_ref[pl.ds(c * CHUNK_B, CHUNK_B), :, :].reshape(GSIZE, HIDDEN)
        y = jnp.dot(xc, w0a, preferred_element_type=jnp.float32)
        h = [y[:, k * HIDDEN:(k + 1) * HIDDEN] for k in range(KGRP)]
        h = [jax.nn.relu(t + b0) for t in agg(h)]
        h = [jnp.dot(t, w1, preferred_element_type=jnp.float32) for t in h]
        h = [jax.nn.relu(t + b1) for t in agg(h)]
        g = agg(h)
        gw = jnp.concatenate(g, axis=1)
        p = jnp.dot(gw, w2s, preferred_element_type=jnp.float32) + b2s
        base = c * GSIZE
        ow = KGRP * GNN_ACT
        mu_ref[pl.ds(base, GSIZE), :] = p[:, :ow]
        ls = jnp.tanh(p[:, ow:])
        ls = LOG_STD_MIN + 0.5 * (LOG_STD_MAX - LOG_STD_MIN) * (ls + 1.0)
        std_ref[pl.ds(base, GSIZE), :] = jnp.exp(ls)
        return carry

    jax.lax.fori_loop(0, nb // CHUNK_B, body, 0, unroll=2)


@functools.partial(jax.jit, static_argnames=())
def kernel(obs, W0, b0, W1, b1, W2, b2):
    bs = obs.shape[0]
    out_w = NUM_NODES * GNN_ACT
    grid = (bs // BATCH_BLOCK,)

    x = obs.reshape(bs, KGRP, HIDDEN)
    eye = jnp.eye(KGRP, dtype=jnp.float32)
    W0all = jnp.kron(eye, W0)
    W2s = jnp.concatenate(
        [jnp.kron(eye, W2[:, :GNN_ACT]), jnp.kron(eye, W2[:, GNN_ACT:])],
        axis=1)
    b2s = jnp.concatenate(
        [jnp.tile(b2[:GNN_ACT], KGRP), jnp.tile(b2[GNN_ACT:], KGRP)])

    ow = KGRP * GNN_ACT
    mu, std = pl.pallas_call(
        _fused_kernel,
        grid=grid,
        in_specs=[
            pl.BlockSpec((BATCH_BLOCK, KGRP, HIDDEN), lambda i: (i, 0, 0)),
            pl.BlockSpec((HIDDEN, KGRP * HIDDEN), lambda i: (0, 0)),
            pl.BlockSpec((1, HIDDEN), lambda i: (0, 0)),
            pl.BlockSpec((HIDDEN, HIDDEN), lambda i: (0, 0)),
            pl.BlockSpec((1, HIDDEN), lambda i: (0, 0)),
            pl.BlockSpec((KGRP * HIDDEN, 2 * KGRP * GNN_ACT), lambda i: (0, 0)),
            pl.BlockSpec((1, 2 * KGRP * GNN_ACT), lambda i: (0, 0)),
        ],
        out_specs=[
            pl.BlockSpec((BATCH_BLOCK * KGRP, ow), lambda i: (i, 0)),
            pl.BlockSpec((BATCH_BLOCK * KGRP, ow), lambda i: (i, 0)),
        ],
        out_shape=[
            jax.ShapeDtypeStruct((bs * KGRP, ow), jnp.float32),
            jax.ShapeDtypeStruct((bs * KGRP, ow), jnp.float32),
        ],
    )(x, W0all, b0.reshape(1, HIDDEN), W1, b1.reshape(1, HIDDEN),
      W2s, b2s.reshape(1, 2 * KGRP * GNN_ACT))

    return (mu.reshape(bs, out_w), std.reshape(bs, out_w))


# speedup vs baseline: 2.2327x; 1.0035x over previous
import functools

import numpy as np
import jax
import jax.numpy as jnp
from jax.experimental import pallas as pl

NUM_NODES = 64
OBS_DIM = 1024
GNN_OBS = OBS_DIM // NUM_NODES
GNN_ACT = 2
HIDDEN = 128
LOG_STD_MIN, LOG_STD_MAX = -5.0, 2.0

BATCH_BLOCK = 512
CHUNK_B = 16
KGRP = 8
GSIZE = CHUNK_B * KGRP
CROWS = CHUNK_B * NUM_NODES


THIRD = 1.0 / 3.0
S6 = float(1.0 / np.sqrt(6.0))


def _edge_coeffs():
    j = jax.lax.rem(jax.lax.broadcasted_iota(jnp.int32, (GSIZE, 1), 0),
                    KGRP)
    j0 = j == 0
    j7 = j == KGRP - 1
    third = jnp.float32(THIRD)
    sel = lambda m, v: jnp.where(m, jnp.float32(v), third)
    av0 = sel(j0, 0.5)
    lv0 = sel(j0, 0.0)
    uv0 = sel(j0, S6)
    lv1 = sel(j0, S6)
    uv6 = sel(j7, S6)
    av7 = sel(j7, 0.5)
    lv7 = sel(j7, S6)
    uv7 = sel(j7, 0.0)
    return av0, lv0, uv0, lv1, uv6, av7, lv7, uv7


def _fused_kernel(x_ref, w0a_ref, b0_ref, w1_ref, b1_ref, w2s_ref, b2s_ref,
                  mu_ref, std_ref):
    nb = x_ref.shape[0]
    av0, lv0, uv0, lv1, uv6, av7, lv7, uv7 = _edge_coeffs()
    w0a = w0a_ref[...]
    b0 = b0_ref[...]
    w1 = w1_ref[...]
    b1 = b1_ref[...]
    w2s = w2s_ref[...]
    b2s = b2s_ref[...]
    third = jnp.float32(THIRD)

    def agg(g):
        prev = [jnp.roll(g[KGRP - 1], 1, axis=0)] + g[:KGRP - 1]
        nxt = g[1:] + [jnp.roll(g[0], -1, axis=0)]
        return [
            av0 * g[0] + lv0 * prev[0] + uv0 * nxt[0],
            third * (g[1] + nxt[1]) + lv1 * prev[1],
            third * ((g[2] + prev[2]) + nxt[2]),
            third * ((g[3] + prev[3]) + nxt[3]),
            third * ((g[4] + prev[4]) + nxt[4]),
            third * ((g[5] + prev[5]) + nxt[5]),
            third * (g[6] + prev[6]) + uv6 * nxt[6],
            av7 * g[7] + lv7 * prev[7] + uv7 * nxt[7],
        ]

    def body(c, carry):
        xc = x_ref[pl.ds(c * CHUNK_B, CHUNK_B), :, :].reshape(GSIZE, HIDDEN)
        y = jnp.dot(xc, w0a, preferred_element_type=jnp.float32)
        h = [y[:, k * HIDDEN:(k + 1) * HIDDEN] for k in range(KGRP)]
        h = [jax.nn.relu(t + b0) for t in agg(h)]
        h = [jnp.dot(t, w1, preferred_element_type=jnp.float32) for t in h]
        h = [jax.nn.relu(t + b1) for t in agg(h)]
        g = agg(h)
        gw = jnp.concatenate(g, axis=1)
        p = jnp.dot(gw, w2s, preferred_element_type=jnp.float32) + b2s
        base = c * GSIZE
        ow = KGRP * GNN_ACT
        mu_ref[pl.ds(base, GSIZE), :] = p[:, :ow]
        ls = jnp.tanh(p[:, ow:])
        ls = LOG_STD_MIN + 0.5 * (LOG_STD_MAX - LOG_STD_MIN) * (ls + 1.0)
        std_ref[pl.ds(base, GSIZE), :] = jnp.exp(ls)
        return carry

    jax.lax.fori_loop(0, nb // CHUNK_B, body, 0, unroll=2)


@functools.partial(jax.jit, static_argnames=())
def kernel(obs, W0, b0, W1, b1, W2, b2):
    bs = obs.shape[0]
    out_w = NUM_NODES * GNN_ACT
    grid = (bs // BATCH_BLOCK,)

    x = obs.reshape(bs, KGRP, HIDDEN)
    eye = jnp.eye(KGRP, dtype=jnp.float32)
    W0all = jnp.kron(eye, W0)
    W2s = jnp.concatenate(
        [jnp.kron(eye, W2[:, :GNN_ACT]), jnp.kron(eye, W2[:, GNN_ACT:])],
        axis=1)
    b2s = jnp.concatenate(
        [jnp.tile(b2[:GNN_ACT], KGRP), jnp.tile(b2[GNN_ACT:], KGRP)])

    ow = KGRP * GNN_ACT
    mu, std = pl.pallas_call(
        _fused_kernel,
        grid=grid,
        in_specs=[
            pl.BlockSpec((BATCH_BLOCK, KGRP, HIDDEN), lambda i: (i, 0, 0)),
            pl.BlockSpec((HIDDEN, KGRP * HIDDEN), lambda i: (0, 0)),
            pl.BlockSpec((1, HIDDEN), lambda i: (0, 0)),
            pl.BlockSpec((HIDDEN, HIDDEN), lambda i: (0, 0)),
            pl.BlockSpec((1, HIDDEN), lambda i: (0, 0)),
            pl.BlockSpec((KGRP * HIDDEN, 2 * KGRP * GNN_ACT), lambda i: (0, 0)),
            pl.BlockSpec((1, 2 * KGRP * GNN_ACT), lambda i: (0, 0)),
        ],
        out_specs=[
            pl.BlockSpec((BATCH_BLOCK * KGRP, ow), lambda i: (i, 0)),
            pl.BlockSpec((BATCH_BLOCK * KGRP, ow), lambda i: (i, 0)),
        ],
        out_shape=[
            jax.ShapeDtypeStruct((bs * KGRP, ow), jnp.float32),
            jax.ShapeDtypeStruct((bs * KGRP, ow), jnp.float32),
        ],
    )(x, W0all, b0.reshape(1, HIDDEN), W1, b1.reshape(1, HIDDEN),
      W2s, b2s.reshape(1, 2 * KGRP * GNN_ACT))

    return (mu.reshape(bs, out_w), std.reshape(bs, out_w))
